# Initial kernel scaffold; baseline (speedup 1.0000x reference)
import jax
import jax.numpy as jnp
from jax.experimental import pallas as pl


def kernel(x, w_enc, w_dec, b_enc, b_pre, stats_last_nonzero):
    raise NotImplementedError("write your pallas kernel here")


# speedup vs baseline: 18.6547x; 18.6547x over previous
import jax
import jax.numpy as jnp
from jax.experimental import pallas as pl
from jax.experimental.pallas import tpu as pltpu

B = 1024
D = 768
H = 32768
K = 128
DEAD_ICUT = 3906

ENC_HT = 1024
DEC_HT = 1024
TK_R = 64


def _ln_body(x_ref, bpre_ref, xp_ref, mu_ref, std_ref):
    x = x_ref[...]
    mu = jnp.mean(x, axis=1, keepdims=True)
    xc = x - mu
    var = jnp.sum(xc * xc, axis=1, keepdims=True) * (1.0 / (D - 1))
    std = jnp.sqrt(var)
    xn = xc / (std + 1e-5)
    xp_ref[...] = xn - bpre_ref[...]
    mu_ref[...] = mu
    std_ref[...] = std


def _enc_body(xp_ref, w_ref, benc_ref, out_ref):
    out_ref[...] = (
        jnp.dot(xp_ref[...], w_ref[...], preferred_element_type=jnp.float32)
        + benc_ref[...]
    )


def _topk_body(stats_ref, pa_ref, lat_ref, ns_ref, nd_ref, cnt_ref):
    i = pl.program_id(0)
    pa = pa_ref[...]
    b = jax.lax.bitcast_convert_type(pa, jnp.int32)
    s = jnp.where(b < 0, b ^ jnp.int32(0x7FFFFFFF), b)

    def step(it, t):
        bit = 30 - it
        cand = t | (jnp.int32(1) << bit.astype(jnp.int32))
        cnt = jnp.sum(
            jnp.where(s >= cand, 1.0, 0.0), axis=1, keepdims=True
        )
        return jnp.where(cnt >= K, cand, t)

    cnt_pos = jnp.sum(jnp.where(s >= 0, 1.0, 0.0), axis=1, keepdims=True)
    t0 = jnp.where(cnt_pos >= K, jnp.int32(0), jnp.int32(-0x80000000))
    t = jax.lax.fori_loop(0, 31, step, t0)

    sel = s >= t
    lat = jnp.where(sel, jnp.maximum(pa, 0.0), 0.0)
    lat_ref[...] = lat

    @pl.when(i == 0)
    def _():
        cnt_ref[...] = jnp.zeros_like(cnt_ref)

    cnt_ref[...] += jnp.sum(jnp.where(lat > 0.0, 1.0, 0.0), axis=0,
                            keepdims=True)

    @pl.when(i == pl.num_programs(0) - 1)
    def _():
        dead = (cnt_ref[...] == 0.0).astype(jnp.int32)
        ns = stats_ref[...] * dead + 1
        ns_ref[...] = ns
        nd_ref[0, 0] = jnp.sum((ns > DEAD_ICUT).astype(jnp.int32))


def _dec_body(lat_ref, wd_ref, bpre_ref, mu_ref, std_ref, out_ref, acc_ref):
    i = pl.program_id(0)

    @pl.when(i == 0)
    def _():
        acc_ref[...] = jnp.zeros_like(acc_ref)

    acc_ref[...] += jnp.dot(lat_ref[...], wd_ref[...],
                            preferred_element_type=jnp.float32)

    @pl.when(i == pl.num_programs(0) - 1)
    def _():
        out_ref[...] = (acc_ref[...] + bpre_ref[...]) * std_ref[...] \
            + mu_ref[...]


def kernel(x, w_enc, w_dec, b_enc, b_pre, stats_last_nonzero):
    f32 = jnp.float32
    bpre2 = b_pre.reshape(1, D)
    benc2 = b_enc.reshape(1, H)
    stats2 = stats_last_nonzero.reshape(1, H)

    xp, mu, std = pl.pallas_call(
        _ln_body,
        out_shape=[
            jax.ShapeDtypeStruct((B, D), f32),
            jax.ShapeDtypeStruct((B, 1), f32),
            jax.ShapeDtypeStruct((B, 1), f32),
        ],
    )(x, bpre2)

    pre = pl.pallas_call(
        _enc_body,
        grid=(H // ENC_HT,),
        in_specs=[
            pl.BlockSpec((B, D), lambda i: (0, 0)),
            pl.BlockSpec((D, ENC_HT), lambda i: (0, i)),
            pl.BlockSpec((1, ENC_HT), lambda i: (0, i)),
        ],
        out_specs=pl.BlockSpec((B, ENC_HT), lambda i: (0, i)),
        out_shape=jax.ShapeDtypeStruct((B, H), f32),
    )(xp, w_enc, benc2)

    lat, ns2, nd = pl.pallas_call(
        _topk_body,
        grid=(B // TK_R,),
        in_specs=[
            pl.BlockSpec((1, H), lambda i: (0, 0)),
            pl.BlockSpec((TK_R, H), lambda i: (i, 0)),
        ],
        out_specs=[
            pl.BlockSpec((TK_R, H), lambda i: (i, 0)),
            pl.BlockSpec((1, H), lambda i: (0, 0)),
            pl.BlockSpec(memory_space=pltpu.SMEM),
        ],
        out_shape=[
            jax.ShapeDtypeStruct((B, H), f32),
            jax.ShapeDtypeStruct((1, H), jnp.int32),
            jax.ShapeDtypeStruct((1, 1), jnp.int32),
        ],
        scratch_shapes=[pltpu.VMEM((1, H), f32)],
    )(stats2, pre)

    rec = pl.pallas_call(
        _dec_body,
        grid=(H // DEC_HT,),
        in_specs=[
            pl.BlockSpec((B, DEC_HT), lambda i: (0, i)),
            pl.BlockSpec((DEC_HT, D), lambda i: (i, 0)),
            pl.BlockSpec((1, D), lambda i: (0, 0)),
            pl.BlockSpec((B, 1), lambda i: (0, 0)),
            pl.BlockSpec((B, 1), lambda i: (0, 0)),
        ],
        out_specs=pl.BlockSpec((B, D), lambda i: (0, 0)),
        out_shape=jax.ShapeDtypeStruct((B, D), f32),
        scratch_shapes=[pltpu.VMEM((B, D), f32)],
    )(lat, w_dec, bpre2, mu, std)

    return (rec, nd[0, 0], lat, ns2.reshape(H))
